# baseline (speedup 1.0000x reference)
import functools

import jax
import jax.numpy as jnp
from jax import lax
from jax.experimental import pallas as pl
from jax.experimental.pallas import tpu as pltpu
from jax.experimental.pallas import tpu_sc as plsc

_NC = 2
_NS = 16
_NW = _NC * _NS
_BPW = 128
_LANES = 16


@functools.lru_cache(maxsize=None)
def _make_gather(seq, word_dim):
    assert word_dim % 8 == 0 and word_dim & (word_dim - 1) == 0
    assert seq % 2 == 0 and seq >= 6
    dq = word_dim // 8
    mesh = plsc.VectorSubcoreMesh(core_axis_name="c", subcore_axis_name="s")

    @functools.partial(
        pl.kernel,
        mesh=mesh,
        compiler_params=pltpu.CompilerParams(
            use_tc_tiling_on_sc=False, needs_layout_passes=False),
        out_type=jax.ShapeDtypeStruct((seq, dq, _NW, 8 * _BPW), jnp.float32),
        scratch_types=[
            pltpu.VMEM((seq, _BPW), jnp.int32),
            pltpu.VMEM((4, _BPW), jnp.int32),
            pltpu.VMEM((_BPW, 2 * word_dim), jnp.float32),
            pltpu.VMEM((_BPW, 2 * word_dim), jnp.float32),
            pltpu.VMEM((dq, 8 * _BPW), jnp.float32),
            pltpu.VMEM((dq, 8 * _BPW), jnp.float32),
            pltpu.SemaphoreType.DMA,
            pltpu.SemaphoreType.DMA,
            pltpu.SemaphoreType.DMA,
            pltpu.SemaphoreType.DMA,
        ],
    )
    def gather_kernel(table_hbm, idx_hbm, out_hbm,
                      idx_v, hidx, g0, g1, t0, t1, gs0, gs1, os0, os1):
        wid = lax.axis_index("s") * _NC + lax.axis_index("c")
        pltpu.sync_copy(idx_hbm.at[wid], idx_v)
        gbuf = (g0, g1)
        tbuf = (t0, t1)
        gsem = (gs0, gs1)
        osem = (os0, os1)

        def g_start(s, b):
            slot = s % 4
            for k in range(_BPW // _LANES):
                v = idx_v[s, pl.ds(k * _LANES, _LANES)]
                hidx[slot, pl.ds(k * _LANES, _LANES)] = v >> 1
            pltpu.async_copy(table_hbm.at[hidx.at[slot]], gbuf[b], gsem[b])

        def g_wait(b):
            pltpu.make_async_copy(
                table_hbm.at[hidx.at[0]], gbuf[b], gsem[b]).wait()

        def w_start(s, b):
            pltpu.async_copy(tbuf[b], out_hbm.at[s, :, wid], osem[b])

        def w_wait(b):
            pltpu.make_async_copy(
                tbuf[b], out_hbm.at[0, :, wid], osem[b]).wait()

        iota = lax.iota(jnp.int32, _LANES)
        rows_c = [iota + (r0 * _LANES) for r0 in range(_BPW // _LANES)]

        def transpose(s, b):
            g, t = gbuf[b], tbuf[b]
            hoff = [
                (idx_v[s, pl.ds(r0 * _LANES, _LANES)] & 1) * word_dim
                for r0 in range(_BPW // _LANES)
            ]

            def dbody(d0, c):
                dvec = (d0 + iota) & (word_dim - 1)
                trow = dvec >> 3
                tcolb = (dvec & 7) << 7
                for r0 in range(_BPW // _LANES):
                    rv = rows_c[r0]
                    vec = plsc.load_gather(g, [rv, dvec + hoff[r0]])
                    plsc.store_scatter(t, [trow, tcolb + rv], vec)
                return c

            lax.fori_loop(0, word_dim, dbody, 0)

        g_start(0, 0)
        g_start(1, 1)

        def body(i, carry):
            s0 = i * 2
            for b in range(2):
                s = s0 + b
                g_wait(b)
                @pl.when(s0 >= 2)
                def _():
                    w_wait(b)
                transpose(s, b)
                w_start(s, b)
                @pl.when(s0 + 2 < seq)
                def _():
                    g_start(s + 2, b)
            return carry

        lax.fori_loop(0, seq // 2, body, 0)
        w_wait(0)
        w_wait(1)

    return gather_kernel


def kernel(sent, W):
    batch, seq = sent.shape
    word_dim = W.shape[1]
    assert batch == _NW * _BPW
    idx = sent.astype(jnp.int32).reshape(_NW, _BPW, seq).transpose(0, 2, 1)
    table = jnp.concatenate([W[0::2], W[1::2]], axis=1)
    y = _make_gather(seq, word_dim)(table, idx)
    y = y.reshape(seq, word_dim // 8, _NW, 8, _BPW)
    return y.transpose(2, 4, 0, 1, 3).reshape(batch, seq, word_dim)


# speedup vs baseline: 8.6048x; 8.6048x over previous
import functools

import jax
import jax.numpy as jnp
from jax import lax
from jax.experimental import pallas as pl
from jax.experimental.pallas import tpu as pltpu
from jax.experimental.pallas import tpu_sc as plsc

_NC = 2
_NS = 16
_NW = _NC * _NS
_BPW = 128
_LANES = 16


@functools.lru_cache(maxsize=None)
def _make_gather(seq, word_dim, n_words):
    assert word_dim % 8 == 0 and word_dim & (word_dim - 1) == 0
    assert seq % 2 == 0 and seq >= 6
    dq = word_dim // 8
    mesh = plsc.VectorSubcoreMesh(core_axis_name="c", subcore_axis_name="s")

    @functools.partial(
        pl.kernel,
        mesh=mesh,
        compiler_params=pltpu.CompilerParams(
            use_tc_tiling_on_sc=False, needs_layout_passes=False),
        out_type=jax.ShapeDtypeStruct((seq, dq, _NW, 8 * _BPW), jnp.float32),
        scratch_types=[
            pltpu.VMEM((seq, _BPW), jnp.int32),
            pltpu.VMEM((_BPW, word_dim), jnp.float32),
            pltpu.VMEM((_BPW, word_dim), jnp.float32),
            pltpu.VMEM((dq, 8 * _BPW), jnp.float32),
            pltpu.VMEM((dq, 8 * _BPW), jnp.float32),
            pltpu.SemaphoreType.DMA,
            pltpu.SemaphoreType.DMA,
            pltpu.SemaphoreType.DMA,
            pltpu.SemaphoreType.DMA,
        ],
    )
    def gather_kernel(table_hbm, idx_hbm, out_hbm,
                      idx_v, g0, g1, t0, t1, gs0, gs1, os0, os1):
        wid = lax.axis_index("s") * _NC + lax.axis_index("c")
        pltpu.sync_copy(idx_hbm.at[wid], idx_v)
        gbuf = (g0, g1)
        tbuf = (t0, t1)
        gsem = (gs0, gs1)
        osem = (os0, os1)

        def g_start(s, b):
            pltpu.async_copy(table_hbm.at[idx_v.at[s]], gbuf[b], gsem[b])

        def g_wait(b):
            pltpu.make_async_copy(
                table_hbm.at[idx_v.at[0]], gbuf[b], gsem[b]).wait()

        def w_start(s, b):
            pltpu.async_copy(tbuf[b], out_hbm.at[s, :, wid], osem[b])

        def w_wait(b):
            pltpu.make_async_copy(
                tbuf[b], out_hbm.at[0, :, wid], osem[b]).wait()

        iota = lax.iota(jnp.int32, _LANES)
        rows_c = [iota + (r0 * _LANES) for r0 in range(_BPW // _LANES)]

        def transpose(b):
            g, t = gbuf[b], tbuf[b]

            def dbody(d0, c):
                dvec = (d0 + iota) & (word_dim - 1)
                trow = dvec >> 3
                tcolb = (dvec & 7) << 7
                for r0 in range(_BPW // _LANES):
                    rv = rows_c[r0]
                    vec = plsc.load_gather(g, [rv, dvec])
                    plsc.store_scatter(t, [trow, tcolb + rv], vec)
                return c

            lax.fori_loop(0, word_dim, dbody, 0, unroll=8)

        g_start(0, 0)
        g_start(1, 1)

        def body(i, carry):
            s0 = i * 2
            for b in range(2):
                s = s0 + b
                g_wait(b)
                @pl.when(s0 >= 2)
                def _():
                    w_wait(b)
                transpose(b)
                w_start(s, b)
                @pl.when(s0 + 2 < seq)
                def _():
                    g_start(s + 2, b)
            return carry

        lax.fori_loop(0, seq // 2, body, 0)
        w_wait(0)
        w_wait(1)

    return gather_kernel


def kernel(sent, W):
    batch, seq = sent.shape
    word_dim = W.shape[1]
    assert batch == _NW * _BPW
    idx = sent.astype(jnp.int32).reshape(_NW, _BPW, seq).transpose(0, 2, 1)
    y = _make_gather(seq, word_dim, W.shape[0])(W, idx)
    y = y.reshape(seq, word_dim // 8, _NW, 8, _BPW)
    return y.transpose(2, 4, 0, 1, 3).reshape(batch, seq, word_dim)


# speedup vs baseline: 11.5834x; 1.3462x over previous
import functools

import jax
import jax.numpy as jnp
from jax import lax
from jax.experimental import pallas as pl
from jax.experimental.pallas import tpu as pltpu
from jax.experimental.pallas import tpu_sc as plsc

_NC = 2
_NS = 16
_NW = _NC * _NS
_BPW = 128
_LANES = 16


@functools.lru_cache(maxsize=None)
def _make_gather(seq, word_dim, n_words):
    assert word_dim % 8 == 0 and word_dim & (word_dim - 1) == 0
    assert seq % 2 == 0 and seq >= 6
    dq = word_dim // 8
    mesh = plsc.VectorSubcoreMesh(core_axis_name="c", subcore_axis_name="s")

    @functools.partial(
        pl.kernel,
        mesh=mesh,
        compiler_params=pltpu.CompilerParams(
            use_tc_tiling_on_sc=False, needs_layout_passes=False),
        out_type=jax.ShapeDtypeStruct((seq, dq, _NW, 8 * _BPW), jnp.float32),
        scratch_types=[
            pltpu.VMEM((seq, _BPW), jnp.int32),
            pltpu.VMEM((_BPW, word_dim), jnp.float32),
            pltpu.VMEM((_BPW, word_dim), jnp.float32),
            pltpu.VMEM((dq, 8 * _BPW), jnp.float32),
            pltpu.VMEM((dq, 8 * _BPW), jnp.float32),
            pltpu.SemaphoreType.DMA,
            pltpu.SemaphoreType.DMA,
            pltpu.SemaphoreType.DMA,
            pltpu.SemaphoreType.DMA,
        ],
    )
    def gather_kernel(table_hbm, idx_hbm, out_hbm,
                      idx_v, g0, g1, t0, t1, gs0, gs1, os0, os1):
        wid = lax.axis_index("s") * _NC + lax.axis_index("c")
        pltpu.sync_copy(idx_hbm.at[wid], idx_v)
        gbuf = (g0, g1)
        tbuf = (t0, t1)
        gsem = (gs0, gs1)
        osem = (os0, os1)

        def g_start(s, b):
            pltpu.async_copy(table_hbm.at[idx_v.at[s]], gbuf[b], gsem[b])

        def g_wait(b):
            pltpu.make_async_copy(
                table_hbm.at[idx_v.at[0]], gbuf[b], gsem[b]).wait()

        def w_start(s, b):
            pltpu.async_copy(tbuf[b], out_hbm.at[s, :, wid], osem[b])

        def w_wait(b):
            pltpu.make_async_copy(
                tbuf[b], out_hbm.at[0, :, wid], osem[b]).wait()

        iota = lax.iota(jnp.int32, _LANES)
        rows_c = [iota + (r0 * _LANES) for r0 in range(_BPW // _LANES)]

        def transpose(b):
            g, t = gbuf[b], tbuf[b]

            def dbody(d0, c):
                dvec = (d0 + iota) & (word_dim - 1)
                trow = dvec >> 3
                tcolb = (dvec & 7) << 7
                vecs = [plsc.load_gather(g, [rows_c[r0], dvec])
                        for r0 in range(_BPW // _LANES)]
                for r0 in range(_BPW // _LANES):
                    plsc.store_scatter(t, [trow, tcolb + rows_c[r0]], vecs[r0])
                return c

            lax.fori_loop(0, word_dim, dbody, 0)

        g_start(0, 0)
        g_start(1, 1)

        def body(i, carry):
            s0 = i * 2
            for b in range(2):
                s = s0 + b
                g_wait(b)
                @pl.when(s0 >= 2)
                def _():
                    w_wait(b)
                transpose(b)
                w_start(s, b)
                @pl.when(s0 + 2 < seq)
                def _():
                    g_start(s + 2, b)
            return carry

        lax.fori_loop(0, seq // 2, body, 0)
        w_wait(0)
        w_wait(1)

    return gather_kernel


def kernel(sent, W):
    batch, seq = sent.shape
    word_dim = W.shape[1]
    assert batch == _NW * _BPW
    idx = sent.astype(jnp.int32).reshape(_NW, _BPW, seq).transpose(0, 2, 1)
    y = _make_gather(seq, word_dim, W.shape[0])(W, idx)
    y = y.reshape(seq, word_dim // 8, _NW, 8, _BPW)
    return y.transpose(2, 4, 0, 1, 3).reshape(batch, seq, word_dim)


# speedup vs baseline: 11.5971x; 1.0012x over previous
import functools

import jax
import jax.numpy as jnp
from jax import lax
from jax.experimental import pallas as pl
from jax.experimental.pallas import tpu as pltpu
from jax.experimental.pallas import tpu_sc as plsc

_NC = 2
_NS = 16
_NW = _NC * _NS
_BPW = 128
_LANES = 16


@functools.lru_cache(maxsize=None)
def _make_gather(seq, word_dim, n_words):
    assert word_dim % 8 == 0 and word_dim & (word_dim - 1) == 0
    assert seq % 2 == 0 and seq >= 6
    dq = word_dim // 8
    mesh = plsc.VectorSubcoreMesh(core_axis_name="c", subcore_axis_name="s")

    @functools.partial(
        pl.kernel,
        mesh=mesh,
        compiler_params=pltpu.CompilerParams(
            use_tc_tiling_on_sc=False, needs_layout_passes=False),
        out_type=jax.ShapeDtypeStruct((seq, dq, _NW, 8 * _BPW), jnp.float32),
        scratch_types=[
            pltpu.VMEM((seq, _BPW), jnp.int32),
            pltpu.VMEM((_BPW, word_dim), jnp.float32),
            pltpu.VMEM((_BPW, word_dim), jnp.float32),
            pltpu.VMEM((dq, 8 * _BPW), jnp.float32),
            pltpu.VMEM((dq, 8 * _BPW), jnp.float32),
            pltpu.SemaphoreType.DMA,
            pltpu.SemaphoreType.DMA,
            pltpu.SemaphoreType.DMA,
            pltpu.SemaphoreType.DMA,
        ],
    )
    def gather_kernel(table_hbm, idx_hbm, out_hbm,
                      idx_v, g0, g1, t0, t1, gs0, gs1, os0, os1):
        wid = lax.axis_index("s") * _NC + lax.axis_index("c")
        pltpu.sync_copy(idx_hbm.at[wid], idx_v)
        gbuf = (g0, g1)
        tbuf = (t0, t1)
        gsem = (gs0, gs1)
        osem = (os0, os1)

        def g_start(s, b):
            pltpu.async_copy(table_hbm.at[idx_v.at[s]], gbuf[b], gsem[b])

        def g_wait(b):
            pltpu.make_async_copy(
                table_hbm.at[idx_v.at[0]], gbuf[b], gsem[b]).wait()

        def w_start(s, b):
            pltpu.async_copy(tbuf[b], out_hbm.at[s, :, wid], osem[b])

        def w_wait(b):
            pltpu.make_async_copy(
                tbuf[b], out_hbm.at[0, :, wid], osem[b]).wait()

        iota = lax.iota(jnp.int32, _LANES)
        rows_c = [iota + (r0 * _LANES) for r0 in range(_BPW // _LANES)]

        def transpose(b):
            g, t = gbuf[b], tbuf[b]

            def dbody(d0, c):
                dvec = (d0 + iota) & (word_dim - 1)
                trow = dvec >> 3
                tcolb = (dvec & 7) << 7
                vecs = [plsc.load_gather(g, [rows_c[r0], dvec])
                        for r0 in range(_BPW // _LANES)]
                for r0 in range(_BPW // _LANES):
                    plsc.store_scatter(t, [trow, tcolb + rows_c[r0]], vecs[r0])
                return c

            lax.fori_loop(0, word_dim, dbody, 0, unroll=2)

        g_start(0, 0)
        g_start(1, 1)

        def body(i, carry):
            s0 = i * 2
            for b in range(2):
                s = s0 + b
                g_wait(b)
                @pl.when(s0 >= 2)
                def _():
                    w_wait(b)
                transpose(b)
                w_start(s, b)
                @pl.when(s0 + 2 < seq)
                def _():
                    g_start(s + 2, b)
            return carry

        lax.fori_loop(0, seq // 2, body, 0)
        w_wait(0)
        w_wait(1)

    return gather_kernel


def kernel(sent, W):
    batch, seq = sent.shape
    word_dim = W.shape[1]
    assert batch == _NW * _BPW
    idx = sent.astype(jnp.int32).reshape(_NW, _BPW, seq).transpose(0, 2, 1)
    y = _make_gather(seq, word_dim, W.shape[0])(W, idx)
    y = y.reshape(seq, word_dim // 8, _NW, 8, _BPW)
    return y.transpose(2, 4, 0, 1, 3).reshape(batch, seq, word_dim)


# speedup vs baseline: 17.8691x; 1.5408x over previous
import functools

import jax
import jax.numpy as jnp
from jax import lax
from jax.experimental import pallas as pl
from jax.experimental.pallas import tpu as pltpu
from jax.experimental.pallas import tpu_sc as plsc

_NC = 2
_NS = 16
_NW = _NC * _NS
_BPW = 128
_LANES = 16


@functools.lru_cache(maxsize=None)
def _make_format(n_words, word_dim):
    assert word_dim == 64
    n_full = n_words // 128
    assert n_words % 128 == 64
    n_loop = (n_full // _NW) & ~1
    n_rem = n_full - n_loop * _NW
    assert n_rem < _NW
    mesh = plsc.VectorSubcoreMesh(core_axis_name="c", subcore_axis_name="s")

    @functools.partial(
        pl.kernel,
        mesh=mesh,
        compiler_params=pltpu.CompilerParams(
            use_tc_tiling_on_sc=True, needs_layout_passes=False),
        out_type=jax.ShapeDtypeStruct((n_words // 2 + 32, 2 * word_dim),
                                      jnp.float32),
        scratch_types=[
            pltpu.VMEM((word_dim, 128), jnp.float32),
            pltpu.VMEM((word_dim, 128), jnp.float32),
            pltpu.VMEM((word_dim, 128), jnp.float32),
            pltpu.VMEM((word_dim, 128), jnp.float32),
            pltpu.VMEM((32, 2 * word_dim), jnp.float32),
            pltpu.SemaphoreType.DMA,
            pltpu.SemaphoreType.DMA,
            pltpu.SemaphoreType.DMA,
            pltpu.SemaphoreType.DMA,
        ],
    )
    def fmt_kernel(wt_hbm, tail_hbm, tbl_hbm,
                   s0, s1, o0, o1, tailv, gs0, gs1, os0, os1):
        wid = lax.axis_index("s") * _NC + lax.axis_index("c")
        sbuf = (s0, s1)
        obuf = (o0, o1)
        gsem = (gs0, gs1)
        osem = (os0, os1)

        def g_start(j, b):
            pltpu.async_copy(
                wt_hbm.at[:, pl.ds((wid + j * _NW) * 128, 128)],
                sbuf[b], gsem[b])

        def g_wait(b):
            pltpu.make_async_copy(
                wt_hbm.at[:, pl.ds(0, 128)], sbuf[b], gsem[b]).wait()

        def w_start(j, b):
            pltpu.async_copy(
                obuf[b], tbl_hbm.at[pl.ds((wid + j * _NW) * 64, 64)], osem[b])

        def w_wait(b):
            pltpu.make_async_copy(
                obuf[b], tbl_hbm.at[pl.ds(0, 64)], osem[b]).wait()

        iota = lax.iota(jnp.int32, _LANES)
        qv = iota >> 1
        hv64 = (iota & 1) << 6

        def transpose(b):
            s, o = sbuf[b], obuf[b]

            def dbody(d0, c):
                dvec = (d0 + iota) & (word_dim - 1)
                dcol = hv64 + dvec
                vecs = [plsc.load_gather(s, [dvec, iota + 2 * q0])
                        for q0 in range(0, 64, 8)]
                for k, q0 in enumerate(range(0, 64, 8)):
                    plsc.store_scatter(o, [qv + q0, dcol], vecs[k])
                return c

            lax.fori_loop(0, word_dim, dbody, 0, unroll=2)

        g_start(0, 0)
        g_start(1, 1)

        def body(i, carry):
            for b in range(2):
                j = i * 2 + b
                g_wait(b)

                @pl.when(i >= 1)
                def _():
                    w_wait(b)
                transpose(b)
                w_start(j, b)

                @pl.when(j + 2 < n_loop)
                def _():
                    g_start(j + 2, b)
            return carry

        lax.fori_loop(0, n_loop // 2, body, 0)

        @pl.when(wid < n_rem)
        def _():
            g_start(n_loop, 0)
            g_wait(0)
            w_wait(0)
            transpose(0)
            w_start(n_loop, 0)

        @pl.when(wid == n_rem)
        def _():
            pltpu.sync_copy(tail_hbm, tailv)
            pltpu.sync_copy(tailv, tbl_hbm.at[pl.ds(n_full * 64, 32)])

        w_wait(0)
        w_wait(1)

    return fmt_kernel


@functools.lru_cache(maxsize=None)
def _make_gather(seq, word_dim, n_rows):
    assert word_dim % 8 == 0 and word_dim & (word_dim - 1) == 0
    assert seq % 2 == 0 and seq >= 6
    dq = word_dim // 8
    mesh = plsc.VectorSubcoreMesh(core_axis_name="c", subcore_axis_name="s")

    @functools.partial(
        pl.kernel,
        mesh=mesh,
        compiler_params=pltpu.CompilerParams(
            use_tc_tiling_on_sc=False, needs_layout_passes=False),
        out_type=jax.ShapeDtypeStruct((seq, dq, _NW, 8 * _BPW), jnp.float32),
        scratch_types=[
            pltpu.VMEM((seq, _BPW), jnp.int32),
            pltpu.VMEM((4, _BPW), jnp.int32),
            pltpu.VMEM((_BPW, 2 * word_dim), jnp.float32),
            pltpu.VMEM((_BPW, 2 * word_dim), jnp.float32),
            pltpu.VMEM((dq, 8 * _BPW), jnp.float32),
            pltpu.VMEM((dq, 8 * _BPW), jnp.float32),
            pltpu.SemaphoreType.DMA,
            pltpu.SemaphoreType.DMA,
            pltpu.SemaphoreType.DMA,
            pltpu.SemaphoreType.DMA,
        ],
    )
    def gather_kernel(table_hbm, idx_hbm, out_hbm,
                      idx_v, hidx, g0, g1, t0, t1, gs0, gs1, os0, os1):
        wid = lax.axis_index("s") * _NC + lax.axis_index("c")
        pltpu.sync_copy(idx_hbm.at[wid], idx_v)
        gbuf = (g0, g1)
        tbuf = (t0, t1)
        gsem = (gs0, gs1)
        osem = (os0, os1)

        def g_start(s, b):
            slot = s % 4
            for k in range(_BPW // _LANES):
                v = idx_v[s, pl.ds(k * _LANES, _LANES)]
                hidx[slot, pl.ds(k * _LANES, _LANES)] = v >> 1
            pltpu.async_copy(table_hbm.at[hidx.at[slot]], gbuf[b], gsem[b])

        def g_wait(b):
            pltpu.make_async_copy(
                table_hbm.at[hidx.at[0]], gbuf[b], gsem[b]).wait()

        def w_start(s, b):
            pltpu.async_copy(tbuf[b], out_hbm.at[s, :, wid], osem[b])

        def w_wait(b):
            pltpu.make_async_copy(
                tbuf[b], out_hbm.at[0, :, wid], osem[b]).wait()

        iota = lax.iota(jnp.int32, _LANES)
        rows_c = [iota + (r0 * _LANES) for r0 in range(_BPW // _LANES)]

        def transpose(s, b):
            g, t = gbuf[b], tbuf[b]
            hoff = [
                (idx_v[s, pl.ds(r0 * _LANES, _LANES)] & 1) * word_dim
                for r0 in range(_BPW // _LANES)
            ]

            def dbody(d0, c):
                dvec = (d0 + iota) & (word_dim - 1)
                trow = dvec >> 3
                tcolb = (dvec & 7) << 7
                vecs = [plsc.load_gather(g, [rows_c[r0], dvec + hoff[r0]])
                        for r0 in range(_BPW // _LANES)]
                for r0 in range(_BPW // _LANES):
                    plsc.store_scatter(t, [trow, tcolb + rows_c[r0]], vecs[r0])
                return c

            lax.fori_loop(0, word_dim, dbody, 0, unroll=2)

        g_start(0, 0)
        g_start(1, 1)

        def body(i, carry):
            s0 = i * 2
            for b in range(2):
                s = s0 + b
                g_wait(b)
                @pl.when(s0 >= 2)
                def _():
                    w_wait(b)
                transpose(s, b)
                w_start(s, b)
                @pl.when(s0 + 2 < seq)
                def _():
                    g_start(s + 2, b)
            return carry

        lax.fori_loop(0, seq // 2, body, 0)
        w_wait(0)
        w_wait(1)

    return gather_kernel


def kernel(sent, W):
    batch, seq = sent.shape
    n_words, word_dim = W.shape
    assert batch == _NW * _BPW
    idx = sent.astype(jnp.int32).reshape(_NW, _BPW, seq).transpose(0, 2, 1)
    tail = W[n_words - n_words % 128:].reshape(-1, 2 * word_dim)
    table = _make_format(n_words, word_dim)(W.T, tail)
    y = _make_gather(seq, word_dim, table.shape[0])(table, idx)
    y = y.reshape(seq, word_dim // 8, _NW, 8, _BPW)
    return y.transpose(2, 4, 0, 1, 3).reshape(batch, seq, word_dim)


# speedup vs baseline: 20.8745x; 1.1682x over previous
import functools

import jax
import jax.numpy as jnp
from jax import lax
from jax.experimental import pallas as pl
from jax.experimental.pallas import tpu as pltpu
from jax.experimental.pallas import tpu_sc as plsc

_NC = 2
_NS = 16
_NW = _NC * _NS
_BPW = 128
_LANES = 16


@functools.lru_cache(maxsize=None)
def _make_format(n_words, word_dim):
    assert word_dim == 64
    n_full = n_words // 128
    assert n_words % 128 == 64
    n_loop = (n_full // _NW) & ~1
    n_rem = n_full - n_loop * _NW
    assert n_rem < _NW
    mesh = plsc.VectorSubcoreMesh(core_axis_name="c", subcore_axis_name="s")

    @functools.partial(
        pl.kernel,
        mesh=mesh,
        compiler_params=pltpu.CompilerParams(
            use_tc_tiling_on_sc=True, needs_layout_passes=False),
        out_type=jax.ShapeDtypeStruct((n_words // 2 + 32, 2 * word_dim),
                                      jnp.float32),
        scratch_types=[
            pltpu.VMEM((word_dim, 128), jnp.float32),
            pltpu.VMEM((word_dim, 128), jnp.float32),
            pltpu.VMEM((word_dim, 128), jnp.float32),
            pltpu.VMEM((word_dim, 128), jnp.float32),
            pltpu.VMEM((32, 2 * word_dim), jnp.float32),
            pltpu.SemaphoreType.DMA,
            pltpu.SemaphoreType.DMA,
            pltpu.SemaphoreType.DMA,
            pltpu.SemaphoreType.DMA,
        ],
    )
    def fmt_kernel(wt_hbm, tail_hbm, tbl_hbm,
                   s0, s1, o0, o1, tailv, gs0, gs1, os0, os1):
        wid = lax.axis_index("s") * _NC + lax.axis_index("c")
        sbuf = (s0, s1)
        obuf = (o0, o1)
        gsem = (gs0, gs1)
        osem = (os0, os1)

        def g_start(j, b):
            pltpu.async_copy(
                wt_hbm.at[:, pl.ds((wid + j * _NW) * 128, 128)],
                sbuf[b], gsem[b])

        def g_wait(b):
            pltpu.make_async_copy(
                wt_hbm.at[:, pl.ds(0, 128)], sbuf[b], gsem[b]).wait()

        def w_start(j, b):
            pltpu.async_copy(
                obuf[b], tbl_hbm.at[pl.ds((wid + j * _NW) * 64, 64)], osem[b])

        def w_wait(b):
            pltpu.make_async_copy(
                obuf[b], tbl_hbm.at[pl.ds(0, 64)], osem[b]).wait()

        iota = lax.iota(jnp.int32, _LANES)
        qv = iota >> 1
        hv64 = (iota & 1) << 6

        def transpose(b):
            s, o = sbuf[b], obuf[b]

            def dbody(d0, c):
                dvec = (d0 + iota) & (word_dim - 1)
                dcol = hv64 + dvec
                vecs = [plsc.load_gather(s, [dvec, iota + 2 * q0])
                        for q0 in range(0, 64, 8)]
                for k, q0 in enumerate(range(0, 64, 8)):
                    plsc.store_scatter(o, [qv + q0, dcol], vecs[k])
                return c

            lax.fori_loop(0, word_dim, dbody, 0, unroll=2)

        g_start(0, 0)
        g_start(1, 1)

        def body(i, carry):
            for b in range(2):
                j = i * 2 + b
                g_wait(b)

                @pl.when(i >= 1)
                def _():
                    w_wait(b)
                transpose(b)
                w_start(j, b)

                @pl.when(j + 2 < n_loop)
                def _():
                    g_start(j + 2, b)
            return carry

        lax.fori_loop(0, n_loop // 2, body, 0)

        @pl.when(wid < n_rem)
        def _():
            g_start(n_loop, 0)
            g_wait(0)
            w_wait(0)
            transpose(0)
            w_start(n_loop, 0)

        @pl.when(wid == n_rem)
        def _():
            pltpu.sync_copy(tail_hbm, tailv)
            pltpu.sync_copy(tailv, tbl_hbm.at[pl.ds(n_full * 64, 32)])

        w_wait(0)
        w_wait(1)

    return fmt_kernel


@functools.lru_cache(maxsize=None)
def _make_gather(seq, word_dim, n_rows):
    assert word_dim % 8 == 0 and word_dim & (word_dim - 1) == 0
    assert seq % 2 == 0 and seq >= 6
    dq = word_dim // 8
    mesh = plsc.VectorSubcoreMesh(core_axis_name="c", subcore_axis_name="s")

    @functools.partial(
        pl.kernel,
        mesh=mesh,
        compiler_params=pltpu.CompilerParams(
            use_tc_tiling_on_sc=False, needs_layout_passes=False),
        out_type=jax.ShapeDtypeStruct((seq, dq, _NW, 8 * _BPW), jnp.float32),
        scratch_types=[
            pltpu.VMEM((seq, _BPW), jnp.int32),
            pltpu.VMEM((_BPW, word_dim), jnp.float32),
            pltpu.VMEM((_BPW, word_dim), jnp.float32),
            pltpu.VMEM((dq, 8 * _BPW), jnp.float32),
            pltpu.VMEM((dq, 8 * _BPW), jnp.float32),
            pltpu.SemaphoreType.DMA,
            pltpu.SemaphoreType.DMA,
            pltpu.SemaphoreType.DMA,
            pltpu.SemaphoreType.DMA,
        ],
    )
    def gather_kernel(table_hbm, idx_hbm, out_hbm,
                      idx_v, g0, g1, t0, t1, gs0, gs1, os0, os1):
        wid = lax.axis_index("s") * _NC + lax.axis_index("c")
        pltpu.sync_copy(idx_hbm.at[wid], idx_v)
        gbuf = (g0, g1)
        tbuf = (t0, t1)
        gsem = (gs0, gs1)
        osem = (os0, os1)

        def g_start(s, b):
            pltpu.async_copy(table_hbm.at[idx_v.at[s]], gbuf[b], gsem[b])

        def g_wait(b):
            pltpu.make_async_copy(
                table_hbm.at[idx_v.at[0]], gbuf[b], gsem[b]).wait()

        def w_start(s, b):
            pltpu.async_copy(tbuf[b], out_hbm.at[s, :, wid], osem[b])

        def w_wait(b):
            pltpu.make_async_copy(
                tbuf[b], out_hbm.at[0, :, wid], osem[b]).wait()

        iota = lax.iota(jnp.int32, _LANES)
        rows_c = [iota + (r0 * _LANES) for r0 in range(_BPW // _LANES)]

        def transpose(s, b):
            g, t = gbuf[b], tbuf[b]

            def dbody(d0, c):
                dvec = (d0 + iota) & (word_dim - 1)
                trow = dvec >> 3
                tcolb = (dvec & 7) << 7
                vecs = [plsc.load_gather(g, [rows_c[r0], dvec])
                        for r0 in range(_BPW // _LANES)]
                for r0 in range(_BPW // _LANES):
                    plsc.store_scatter(t, [trow, tcolb + rows_c[r0]], vecs[r0])
                return c

            lax.fori_loop(0, word_dim, dbody, 0, unroll=2)

        g_start(0, 0)
        g_start(1, 1)

        def body(i, carry):
            s0 = i * 2
            for b in range(2):
                s = s0 + b
                g_wait(b)
                @pl.when(s0 >= 2)
                def _():
                    w_wait(b)
                transpose(s, b)
                w_start(s, b)
                @pl.when(s0 + 2 < seq)
                def _():
                    g_start(s + 2, b)
            return carry

        lax.fori_loop(0, seq // 2, body, 0)
        w_wait(0)
        w_wait(1)

    return gather_kernel


def kernel(sent, W):
    batch, seq = sent.shape
    n_words, word_dim = W.shape
    assert batch == _NW * _BPW
    idx = sent.astype(jnp.int32).reshape(_NW, _BPW, seq).transpose(0, 2, 1)
    tail = W[n_words - n_words % 128:].reshape(-1, 2 * word_dim)
    table = _make_format(n_words, word_dim)(W.T, tail)
    table = table.reshape(-1, word_dim)
    y = _make_gather(seq, word_dim, table.shape[0])(table, idx)
    y = y.reshape(seq, word_dim // 8, _NW, 8, _BPW)
    return y.transpose(2, 4, 0, 1, 3).reshape(batch, seq, word_dim)
